# baseline (speedup 1.0000x reference)
import jax
import jax.numpy as jnp
from jax import lax
from jax.experimental import pallas as pl
from jax.experimental.pallas import tpu as pltpu
from jax.experimental.pallas import tpu_sc as plsc

_B, _S, _H = 4, 2048, 4096
_HH = _H // 4
_NC, _NS = 2, 16
_NW = _NC * _NS
_NPG = _NW // 4
_PW = _S // _NPG
_W = 8
_NPC = _PW // _W
_D = 3
_LANES = 16
_UNROLL = 8


def _add_pos4(r0, r1, r2, r3, pos):
    @pl.loop(0, _W)
    def _(r):
        @pl.loop(0, _HH, step=_LANES * _UNROLL)
        def _(c):
            vals = [pos[r, pl.ds(c + _LANES * u, _LANES)] for u in range(_UNROLL)]
            for u in range(_UNROLL):
                for rb in (r0, r1, r2, r3):
                    plsc.addupdate(rb.at[r, pl.ds(c + _LANES * u, _LANES)], vals[u])


def _body(x_hbm, tok_hbm, pos_hbm, out_hbm, *scr):
    idx_v = scr[0]
    pos = scr[1:1 + _D]
    rflat = scr[1 + _D:1 + _D + _D * _B]
    rows = tuple(rflat[d * _B:(d + 1) * _B] for d in range(_D))
    o = 1 + _D + _D * _B
    gflat = scr[o:o + _D * _B]
    gsem = tuple(gflat[d * _B:(d + 1) * _B] for d in range(_D))
    o += _D * _B
    sflat = scr[o:o + _D * _B]
    ssem = tuple(sflat[d * _B:(d + 1) * _B] for d in range(_D))
    o += _D * _B
    psem = scr[o:o + _D]

    wid = lax.axis_index("c") * _NS + lax.axis_index("s")
    h = wid & 3
    pg = wid >> 2
    p0 = pg * _PW
    c0 = h * _HH

    for b in range(_B):
        pltpu.sync_copy(x_hbm.at[pl.ds(b * _S + p0, _PW)],
                        idx_v.at[pl.ds(b * _PW, _PW)])

    def g_desc(pc, b, d):
        return pltpu.make_async_copy(
            tok_hbm.at[idx_v.at[pl.ds(b * _PW + pc * _W, _W)], pl.ds(c0, _HH)],
            rows[d][b], gsem[d][b])

    def s_desc(pc, b, d):
        return pltpu.make_async_copy(
            rows[d][b],
            out_hbm.at[pl.ds(b * _S + p0 + pc * _W, _W), pl.ds(c0, _HH)],
            ssem[d][b])

    def p_desc(pc, pd):
        return pltpu.make_async_copy(
            pos_hbm.at[pl.ds(p0 + pc * _W, _W), pl.ds(c0, _HH)],
            pos[pd], psem[pd])

    def chunk(pc, d, guard_drain, prefetch):
        for b in range(_B):
            g_desc(pc, b, d).wait()
        p_desc(pc, d).wait()
        _add_pos4(rows[d][0], rows[d][1], rows[d][2], rows[d][3], pos[d])
        for b in range(_B):
            s_desc(pc, b, d).start()

        d2 = (d + 2) % _D

        def drain():
            for b in range(_B):
                s_desc(pc - 1, b, d2).wait()

        if guard_drain:
            pl.when(pc > 0)(drain)
        else:
            drain()
        if prefetch:
            for b in range(_B):
                g_desc(pc + 2, b, d2).start()
            p_desc(pc + 2, d2).start()

    p_desc(0, 0).start()
    p_desc(1, 1).start()
    for b in range(_B):
        g_desc(0, b, 0).start()
    for b in range(_B):
        g_desc(1, b, 1).start()

    @pl.loop(0, _NPC - 2, step=_D)
    def _(pc):
        chunk(pc, 0, True, True)
        chunk(pc + 1, 1, False, True)
        chunk(pc + 2, 2, False, True)

    chunk(_NPC - 2, 0, False, False)
    chunk(_NPC - 1, 1, False, False)
    for b in range(_B):
        s_desc(_NPC - 1, b, 1).wait()


_emb_call = pl.kernel(
    _body,
    out_type=jax.ShapeDtypeStruct((_B * _S, _H), jnp.float32),
    mesh=plsc.VectorSubcoreMesh(core_axis_name="c", subcore_axis_name="s"),
    scratch_types=(
        [pltpu.VMEM((_B * _PW,), jnp.int32)]
        + [pltpu.VMEM((_W, _HH), jnp.float32) for _ in range(_D)]
        + [pltpu.VMEM((_W, _HH), jnp.float32) for _ in range(_D * _B)]
        + [pltpu.SemaphoreType.DMA for _ in range(_D * _B)]
        + [pltpu.SemaphoreType.DMA for _ in range(_D * _B)]
        + [pltpu.SemaphoreType.DMA for _ in range(_D)]
    ),
)


@jax.jit
def _emb(x_flat, tok_emb, pos_emb):
    return _emb_call(x_flat, tok_emb, pos_emb)


def kernel(x, tok_emb, pos_emb):
    x_flat = x.reshape(-1).astype(jnp.int32)
    out = _emb(x_flat, tok_emb, pos_emb)
    return out.reshape(_B, _S, _H)


# speedup vs baseline: 1.0233x; 1.0233x over previous
import jax
import jax.numpy as jnp
from jax import lax
from jax.experimental import pallas as pl
from jax.experimental.pallas import tpu as pltpu
from jax.experimental.pallas import tpu_sc as plsc

_B, _S, _H = 4, 2048, 4096
_HH = _H // 4
_NC, _NS = 2, 16
_NW = _NC * _NS
_NPG = _NW // 4
_PW = _S // _NPG
_W = 8
_CR = _B * _W
_NPC = _PW // _W
_LANES = 16
_UNROLL = 8


def _add_pos4(rows, pos):
    @pl.loop(0, _W)
    def _(r):
        @pl.loop(0, _HH, step=_LANES * _UNROLL)
        def _(c):
            vals = [pos[r, pl.ds(c + _LANES * u, _LANES)] for u in range(_UNROLL)]
            for u in range(_UNROLL):
                for b in range(_B):
                    plsc.addupdate(rows.at[b * _W + r, pl.ds(c + _LANES * u, _LANES)],
                                   vals[u])


def _body(x_hbm, tok_hbm, pos_hbm, out_hbm,
          idx_v, pos0, pos1, rows0, rows1,
          gsem0, gsem1,
          sa0, sb0, sc0, sd0, sa1, sb1, sc1, sd1,
          psem0, psem1):
    wid = lax.axis_index("c") * _NS + lax.axis_index("s")
    h = wid & 3
    pg = wid >> 2
    p0 = pg * _PW
    c0 = h * _HH
    rows = (rows0, rows1)
    gsem = (gsem0, gsem1)
    ssem = ((sa0, sb0, sc0, sd0), (sa1, sb1, sc1, sd1))
    pos = (pos0, pos1)
    psem = (psem0, psem1)

    def p_desc(pc, pd):
        return pltpu.make_async_copy(
            pos_hbm.at[pl.ds(p0 + pc * _W, _W), pl.ds(c0, _HH)],
            pos[pd], psem[pd])

    p_desc(0, 0).start()
    p_desc(1, 1).start()

    pltpu.sync_copy(x_hbm.at[pl.ds(pg * _NPC * _CR, _NPC * _CR)], idx_v)

    def g_desc(pc, d):
        return pltpu.make_async_copy(
            tok_hbm.at[idx_v.at[pl.ds(pc * _CR, _CR)], pl.ds(c0, _HH)],
            rows[d], gsem[d])

    def s_desc(pc, b, d):
        return pltpu.make_async_copy(
            rows[d].at[pl.ds(b * _W, _W), :],
            out_hbm.at[pl.ds(b * _S + p0 + pc * _W, _W), pl.ds(c0, _HH)],
            ssem[d][b])

    def chunk(pc, d, guard_drain, guard_gather):
        def drain():
            for b in range(_B):
                s_desc(pc - 1, b, d ^ 1).wait()

        def prefetch():
            g_desc(pc + 1, d ^ 1).start()

        if guard_drain:
            pl.when(pc > 0)(drain)
        else:
            drain()
        if guard_gather:
            pl.when(pc + 1 < _NPC)(prefetch)
        else:
            prefetch()
        g_desc(pc, d).wait()
        p_desc(pc, d).wait()
        _add_pos4(rows[d], pos[d])
        for b in range(_B):
            s_desc(pc, b, d).start()

    g_desc(0, 0).start()

    @pl.loop(0, _NPC, step=2)
    def _(pc):
        chunk(pc, 0, True, False)

        @pl.when(pc + 2 < _NPC)
        def _():
            p_desc(pc + 2, 0).start()
        chunk(pc + 1, 1, False, True)

        @pl.when(pc + 3 < _NPC)
        def _():
            p_desc(pc + 3, 1).start()

    for b in range(_B):
        s_desc(_NPC - 1, b, 1).wait()


_emb_call = pl.kernel(
    _body,
    out_type=jax.ShapeDtypeStruct((_B * _S, _H), jnp.float32),
    mesh=plsc.VectorSubcoreMesh(core_axis_name="c", subcore_axis_name="s"),
    scratch_types=[
        pltpu.VMEM((_NPC * _CR,), jnp.int32),
        pltpu.VMEM((_W, _HH), jnp.float32),
        pltpu.VMEM((_W, _HH), jnp.float32),
        pltpu.VMEM((_CR, _HH), jnp.float32),
        pltpu.VMEM((_CR, _HH), jnp.float32),
        pltpu.SemaphoreType.DMA,
        pltpu.SemaphoreType.DMA,
        pltpu.SemaphoreType.DMA,
        pltpu.SemaphoreType.DMA,
        pltpu.SemaphoreType.DMA,
        pltpu.SemaphoreType.DMA,
        pltpu.SemaphoreType.DMA,
        pltpu.SemaphoreType.DMA,
        pltpu.SemaphoreType.DMA,
        pltpu.SemaphoreType.DMA,
        pltpu.SemaphoreType.DMA,
        pltpu.SemaphoreType.DMA,
    ],
)


@jax.jit
def _emb(x_t, tok_emb, pos_emb):
    return _emb_call(x_t, tok_emb, pos_emb)


def kernel(x, tok_emb, pos_emb):
    x_t = (x.astype(jnp.int32)
           .reshape(_B, _S // _W, _W)
           .transpose(1, 0, 2)
           .reshape(-1))
    out = _emb(x_t, tok_emb, pos_emb)
    return out.reshape(_B, _S, _H)
